# baseline (speedup 1.0000x reference)
import functools

import jax
import jax.numpy as jnp
from jax import lax
from jax.experimental import pallas as pl
from jax.experimental.pallas import tpu as pltpu
from jax.experimental.pallas import tpu_sc as plsc

B, N, H, E = 4, 512, 32, 16
NC, NS, L = 2, 16, 16
NW = NC * NS
JW = N // NW
BI = 64
NB = N // BI


def _sc_body(edge_hbm, condt_hbm, out_hbm, ct_v, cond_v, ebuf0, ebuf1,
             accv, sem0, sem1):
    wid = lax.axis_index("s") * NC + lax.axis_index("c")
    j0 = wid * JW

    pltpu.sync_copy(condt_hbm.at[wid], ct_v)
    jiota = lax.iota(jnp.int32, L)

    def cond_step(i, _):
        cond_v[i, :] = plsc.load_gather(ct_v, [jiota, jnp.full((L,), i, jnp.int32)])
        return 0
    lax.fori_loop(0, N, cond_step, 0)

    ebufs = (ebuf0, ebuf1)
    sems = (sem0, sem1)

    for b in range(B):
        pending = [None, None]
        pending[0] = pltpu.async_copy(
            edge_hbm.at[b, pl.ds(0, BI), pl.ds(j0, JW), :], ebufs[0], sems[0])
        acc = tuple(jnp.zeros((L,), jnp.float32) for _ in range(E))
        for ib in range(NB):
            cur = ib & 1
            if ib + 1 < NB:
                pending[1 - cur] = pltpu.async_copy(
                    edge_hbm.at[b, pl.ds((ib + 1) * BI, BI), pl.ds(j0, JW), :],
                    ebufs[1 - cur], sems[1 - cur])
            pending[cur].wait()
            eb = ebufs[cur]

            def acc_step(il, a, _ib=ib, _eb=eb):
                i = _ib * BI + il
                cv = cond_v[i, :]
                iv = jnp.full((L,), il, jnp.int32)
                out = []
                for e in range(E):
                    ev = plsc.load_gather(
                        _eb, [iv, jiota, jnp.full((L,), e, jnp.int32)])
                    out.append(a[e] + cv * ev)
                return tuple(out)

            acc = lax.fori_loop(0, BI, acc_step, acc)
        for e in range(E):
            f = (b * E + e) * JW
            accv[f // 128, pl.ds(f % 128, JW)] = acc[e]

    pltpu.sync_copy(accv, out_hbm.at[wid])


@functools.lru_cache(maxsize=1)
def _sc_edge_reduce():
    return pl.kernel(
        _sc_body,
        out_type=jax.ShapeDtypeStruct((NW, 8, 128), jnp.float32),
        mesh=plsc.VectorSubcoreMesh(core_axis_name="c", subcore_axis_name="s",
                                    num_cores=NC, num_subcores=NS),
        compiler_params=pltpu.CompilerParams(needs_layout_passes=False,
                                             use_tc_tiling_on_sc=False),
        scratch_types=[
            pltpu.VMEM((JW, N), jnp.float32),
            pltpu.VMEM((N, JW), jnp.float32),
            pltpu.VMEM((BI, JW, E), jnp.float32),
            pltpu.VMEM((BI, JW, E), jnp.float32),
            pltpu.VMEM((8, 128), jnp.float32),
            pltpu.SemaphoreType.DMA,
            pltpu.SemaphoreType.DMA,
        ],
    )


def _tc_pre_body(h_ref, adj_ref, Ww_ref, Wb_ref, condt_ref, mh_ref):
    c = adj_ref[0] + adj_ref[1] + adj_ref[2] + adj_ref[3]
    cond = jnp.where(c > 0.0, 1.0, 0.0).astype(jnp.float32)
    condt_ref[:] = jnp.transpose(cond).reshape(NW, JW, N)
    Wh = Ww_ref[:, :H]
    Wb = Wb_ref[:]
    for b in range(B):
        A = lax.dot_general(h_ref[b], Wh, (((1,), (1,)), ((), ()))) + Wb
        mh_ref[b] = lax.dot_general(cond, A, (((0,), (0,)), ((), ())))


def _tc_pre(h, adj, W_w, W_b):
    return pl.pallas_call(
        _tc_pre_body,
        out_shape=(jax.ShapeDtypeStruct((NW, JW, N), jnp.float32),
                   jax.ShapeDtypeStruct((B, N, H), jnp.float32)),
        in_specs=[pl.BlockSpec(memory_space=pltpu.VMEM)] * 4,
        out_specs=(pl.BlockSpec(memory_space=pltpu.VMEM),
                   pl.BlockSpec(memory_space=pltpu.VMEM)),
    )(h, adj, W_w, W_b)


def _tc_post_body(nn_ref, h_ref, mh_ref, s2_ref, Ww_ref, Uw_ref, Ub_ref,
                  out_ref):
    We = Ww_ref[:, H:]
    Ub = Ub_ref[:]
    for b in range(B):
        hb = h_ref[b]
        Me = lax.dot_general(s2_ref[b], We, (((1,), (1,)), ((), ())))
        msg = mh_ref[b] + Me
        mask = (lax.broadcasted_iota(jnp.int32, (N, 1), 0)
                < nn_ref[b]).astype(jnp.float32)
        out_ref[b] = lax.dot_general(hb + msg * mask, Uw_ref[:],
                                     (((1,), (1,)), ((), ()))) + Ub


def _tc_post(num_nodes, h, mh, s2, W_w, U_w, U_b):
    return pl.pallas_call(
        _tc_post_body,
        out_shape=jax.ShapeDtypeStruct((B, N, H), jnp.float32),
        in_specs=[pl.BlockSpec(memory_space=pltpu.SMEM)]
        + [pl.BlockSpec(memory_space=pltpu.VMEM)] * 6,
        out_specs=pl.BlockSpec(memory_space=pltpu.VMEM),
    )(num_nodes, h, mh, s2, W_w, U_w, U_b)


def kernel(h, edge_attr, adj, num_nodes, W_w, W_b, U_w, U_b):
    condt, mh = _tc_pre(h, adj, W_w, W_b.reshape(1, H))
    s2raw = _sc_edge_reduce()(edge_attr, condt)
    s2 = (s2raw.reshape(NW, B, E, JW).transpose(1, 0, 3, 2)
          .reshape(B, N, E))
    return _tc_post(num_nodes.astype(jnp.int32), h, mh, s2,
                    W_w, U_w, U_b.reshape(1, H))
